# baseline (speedup 1.0000x reference)
import jax
import jax.numpy as jnp
from jax import lax
from jax.experimental import pallas as pl
from jax.experimental.pallas import tpu as pltpu
from jax.experimental.pallas import tpu_sc as plsc

L = 200
B = 1024
M = 64
MH = M // 2
MP = 2 * M
BLK = 128
BPR = B // BLK
NROWS_STAGE = 8


def _make_body(shifts):
    def _body(t1a, t1b, idx1, t2a, t2b, idx2, zeros_hbm,
              out1, out2, idx_v, rows_v, sem_g, sem_w):
        cid = lax.axis_index("c")
        sid = lax.axis_index("s")
        wid = sid * 2 + cid
        t0 = jnp.where(wid < 24, 6 * wid, 7 * wid - 24)
        nblk = jnp.where(wid < 24, 6 * BPR, 7 * BPR)
        start = jnp.where(wid == 0, BPR, 0)

        for k, (ta, tb, idx, out, shifted) in enumerate(
            ((t1a, t1b, idx1, out1, shifts[0]),
             (t2a, t2b, idx2, out2, shifts[1]))):
            if shifted:
                rl = jnp.clip(t0 - 1, 0, L - NROWS_STAGE)
                dr = t0 - 1 - rl
            else:
                rl = jnp.minimum(t0, L - NROWS_STAGE)
                dr = t0 - rl

            pltpu.sync_copy(idx.at[pl.ds(rl, NROWS_STAGE)], idx_v.at[k])

            def fire_gathers(j, b, ta=ta, tb=tb, k=k, dr=dr):
                r = lax.div(j, BPR) + dr
                c = lax.rem(j, BPR) * BLK
                win = idx_v.at[k].at[r].at[pl.ds(c, BLK)]
                pltpu.async_copy(ta.at[win], rows_v.at[b].at[0], sem_g)
                pltpu.async_copy(tb.at[win], rows_v.at[b].at[1], sem_g)

            def fire_writes(j, b, out=out):
                t = t0 + lax.div(j, BPR)
                c = lax.rem(j, BPR) * BLK
                dst = out.at[t].at[pl.ds(c, BLK)]
                pltpu.async_copy(rows_v.at[b].at[0],
                                 dst.at[:, pl.ds(0, MH)], sem_w)
                pltpu.async_copy(rows_v.at[b].at[1],
                                 dst.at[:, pl.ds(MH, MH)], sem_w)

            def drain_gathers(b, out=out):
                for h in range(2):
                    pltpu.make_async_copy(
                        out.at[0].at[pl.ds(0, BLK), pl.ds(0, MH)],
                        rows_v.at[b].at[h], sem_g).wait()

            def drain_writes(b, out=out):
                for h in range(2):
                    pltpu.make_async_copy(
                        rows_v.at[b].at[h],
                        out.at[0].at[pl.ds(0, BLK), pl.ds(0, MH)],
                        sem_w).wait()

            fire_gathers(start, start % 2)

            def body(j, carry):
                b = lax.rem(j, 2)

                @pl.when(j >= start + 1)
                def _():
                    drain_writes(lax.rem(j - 1, 2))

                @pl.when(j + 1 < nblk)
                def _():
                    fire_gathers(j + 1, lax.rem(j + 1, 2))

                drain_gathers(b)
                fire_writes(j, b)
                return carry

            lax.fori_loop(start, nblk, body, 0)
            drain_writes(lax.rem(nblk - 1, 2))

        @pl.when(wid == 0)
        def _():
            for out in (out1, out2):
                pltpu.sync_copy(zeros_hbm, out.at[0].at[:, pl.ds(0, M)])

    return _body


def _gather_call2(t1, idx1, t2, idx2, zeros_hbm, shifts):
    mesh = plsc.VectorSubcoreMesh(core_axis_name="c", subcore_axis_name="s")
    f = pl.kernel(
        _make_body(shifts),
        out_type=[jax.ShapeDtypeStruct((L, B, MP), jnp.float32)] * 2,
        mesh=mesh,
        scratch_types=[
            pltpu.VMEM((2, NROWS_STAGE, B), jnp.int32),
            pltpu.VMEM((2, 2, BLK, MH), jnp.float32),
            pltpu.SemaphoreType.DMA,
            pltpu.SemaphoreType.DMA,
        ],
        compiler_params=pltpu.CompilerParams(use_tc_tiling_on_sc=False),
    )
    o1, o2 = f(t1[:, :MH], t1[:, MH:], idx1, t2[:, :MH], t2[:, MH:], idx2,
               zeros_hbm)
    return o1[:, :, :M], o2[:, :, :M]


@jax.jit
def kernel(ly, lp, ry, re, W_emb, W_re, pos_emb):
    zeros_hbm = jnp.zeros((B, M), jnp.float32)
    out_e, out_p = _gather_call2(
        W_re, re.astype(jnp.int32), pos_emb, lp.astype(jnp.int32),
        zeros_hbm, (False, True))
    out_l, out_r = _gather_call2(
        W_emb, ly.astype(jnp.int32), W_emb, ry.astype(jnp.int32),
        zeros_hbm, (True, False))
    return (out_l, out_p, out_r, out_e)


# speedup vs baseline: 1.9292x; 1.9292x over previous
import jax
import jax.numpy as jnp
from jax import lax
from jax.experimental import pallas as pl
from jax.experimental.pallas import tpu as pltpu
from jax.experimental.pallas import tpu_sc as plsc

L = 200
B = 1024
M = 64
MP = 2 * M
BLK = 128
BPR = B // BLK
NROWS_STAGE = 8


def _make_body(shifts):
    def _body(table1, idx1, table2, idx2, zeros_hbm, out1, out2,
              idx_v, rows_v, sem_g, sem_w):
        cid = lax.axis_index("c")
        sid = lax.axis_index("s")
        wid = sid * 2 + cid
        t0 = jnp.where(wid < 24, 6 * wid, 7 * wid - 24)
        nblk = jnp.where(wid < 24, 6 * BPR, 7 * BPR)
        start = jnp.where(wid == 0, BPR, 0)

        for k, (table, idx, out, shifted) in enumerate(
            ((table1, idx1, out1, shifts[0]), (table2, idx2, out2, shifts[1]))):
            if shifted:
                rl = jnp.clip(t0 - 1, 0, L - NROWS_STAGE)
                dr = t0 - 1 - rl
            else:
                rl = jnp.minimum(t0, L - NROWS_STAGE)
                dr = t0 - rl

            pltpu.sync_copy(idx.at[pl.ds(rl, NROWS_STAGE)], idx_v.at[k])

            def fire_gather(j, b, table=table, k=k, dr=dr):
                r = lax.div(j, BPR) + dr
                c = lax.rem(j, BPR) * BLK
                pltpu.async_copy(
                    table.at[idx_v.at[k].at[r].at[pl.ds(c, BLK)]],
                    rows_v.at[b], sem_g)

            def fire_write(j, b, out=out):
                t = t0 + lax.div(j, BPR)
                c = lax.rem(j, BPR) * BLK
                pltpu.async_copy(
                    rows_v.at[b], out.at[t].at[pl.ds(c, BLK), pl.ds(0, M)],
                    sem_w)

            def drain_gather(b, out=out):
                pltpu.make_async_copy(
                    out.at[0].at[pl.ds(0, BLK), pl.ds(0, M)],
                    rows_v.at[b], sem_g).wait()

            def drain_write(b, out=out):
                pltpu.make_async_copy(
                    rows_v.at[b],
                    out.at[0].at[pl.ds(0, BLK), pl.ds(0, M)], sem_w).wait()

            fire_gather(start, start % 2)

            def body(j, carry):
                b = lax.rem(j, 2)

                @pl.when(j >= start + 1)
                def _():
                    drain_write(lax.rem(j - 1, 2))

                @pl.when(j + 1 < nblk)
                def _():
                    fire_gather(j + 1, lax.rem(j + 1, 2))

                drain_gather(b)
                fire_write(j, b)
                return carry

            lax.fori_loop(start, nblk, body, 0)
            drain_write(lax.rem(nblk - 1, 2))

        @pl.when(wid == 0)
        def _():
            for out in (out1, out2):
                pltpu.sync_copy(zeros_hbm, out.at[0].at[:, pl.ds(0, M)])

    return _body


def _gather_call2(table1, idx1, table2, idx2, zeros_hbm, shifts):
    mesh = plsc.VectorSubcoreMesh(core_axis_name="c", subcore_axis_name="s")
    f = pl.kernel(
        _make_body(shifts),
        out_type=[jax.ShapeDtypeStruct((L, B, MP), jnp.float32)] * 2,
        mesh=mesh,
        scratch_types=[
            pltpu.VMEM((2, NROWS_STAGE, B), jnp.int32),
            pltpu.VMEM((2, BLK, M), jnp.float32),
            pltpu.SemaphoreType.DMA,
            pltpu.SemaphoreType.DMA,
        ],
        compiler_params=pltpu.CompilerParams(use_tc_tiling_on_sc=False),
    )
    o1, o2 = f(table1, idx1, table2, idx2, zeros_hbm)
    return o1[:, :, :M], o2[:, :, :M]


@jax.jit
def kernel(ly, lp, ry, re, W_emb, W_re, pos_emb):
    zeros_hbm = jnp.zeros((B, M), jnp.float32)
    out_e, out_p = _gather_call2(
        W_re, re.astype(jnp.int32), pos_emb, lp.astype(jnp.int32),
        zeros_hbm, (False, True))
    out_l, out_r = _gather_call2(
        W_emb, ly.astype(jnp.int32), W_emb, ry.astype(jnp.int32),
        zeros_hbm, (True, False))
    return (out_l, out_p, out_r, out_e)
